# baseline (speedup 1.0000x reference)
import jax
import jax.numpy as jnp
from jax.experimental import pallas as pl
from jax.experimental.pallas import tpu as pltpu

D = 128
H = 8
DK = 16
FF = 512
LE = 3
LD = 3
KE = 100
KS = 100
NTOK = 1 + KS + KE + KE
NDEC = 1 + KS + KE


def _topk_min_idx(vals, k, length):
    iota = jax.lax.broadcasted_iota(jnp.int32, (1, length), 1)
    iotak = jax.lax.broadcasted_iota(jnp.int32, (1, k), 1)

    def body(i, carry):
        v, idxs = carry
        m = jnp.min(v)
        idx = jnp.min(jnp.where(v == m, iota, length))
        v = jnp.where(iota == idx, jnp.float32(jnp.inf), v)
        idxs = jnp.where(iotak == i, idx, idxs)
        return v, idxs

    _, idxs = jax.lax.fori_loop(
        0, k, body, (vals, jnp.zeros((1, k), jnp.int32)))
    return idxs


def _prep_kernel(parT_ref, unsT_ref, curT_ref, allc_ref, sidx_ref):
    parT = parT_ref[0]
    unsT = unsT_ref[0]
    curT = curT_ref[0]
    p = parT.shape[1]
    up = unsT.shape[1]

    dpar = parT - curT
    d1 = jnp.sqrt(jnp.sum(dpar * dpar, axis=0, keepdims=True))
    d2 = jnp.concatenate([d1[:, 1:], d1[:, :1]], axis=1)
    dmin = jnp.minimum(d1, d2)
    sidx = _topk_min_idx(dmin, KE, p)

    dun = unsT - curT
    d3 = jnp.sqrt(jnp.sum(dun * dun, axis=0, keepdims=True))
    sidx2 = _topk_min_idx(d3, KS, up)

    ohT = (jax.lax.broadcasted_iota(jnp.int32, (p, KE), 0)
           == sidx).astype(jnp.float32)
    left1T = jnp.dot(parT, ohT, preferred_element_type=jnp.float32)
    par2T = jnp.concatenate([parT[:, 1:], parT[:, :1]], axis=1)
    left2T = jnp.dot(par2T, ohT, preferred_element_type=jnp.float32)
    ohT2 = (jax.lax.broadcasted_iota(jnp.int32, (up, KS), 0)
            == sidx2).astype(jnp.float32)
    unskT = jnp.dot(unsT, ohT2, preferred_element_type=jnp.float32)

    allc = jnp.concatenate([curT, unskT, left1T, left2T], axis=1)
    mn = jnp.min(allc, axis=1, keepdims=True)
    mx = jnp.max(allc, axis=1, keepdims=True)
    allc = (allc - mn) / (mx - mn)

    allc_ref[0] = allc
    sidx_ref[0] = sidx


def _layer_tc(x, Wq, Wk, Wv, Wc, bc, W1, b1, W2, b2):
    q = jnp.dot(x, Wq, preferred_element_type=jnp.float32)
    k = jnp.dot(x, Wk, preferred_element_type=jnp.float32)
    v = jnp.dot(x, Wv, preferred_element_type=jnp.float32)
    outs = []
    for h in range(H):
        qh = q[:, h * DK:(h + 1) * DK]
        kh = k[:, h * DK:(h + 1) * DK]
        vh = v[:, h * DK:(h + 1) * DK]
        s = jnp.dot(qh, kh.T, preferred_element_type=jnp.float32) * 0.25
        s = s - jnp.max(s, axis=1, keepdims=True)
        e = jnp.exp(s)
        w = e / jnp.sum(e, axis=1, keepdims=True)
        outs.append(jnp.dot(w, vh, preferred_element_type=jnp.float32))
    oc = jnp.concatenate(outs, axis=1)
    o1 = x + jnp.dot(oc, Wc, preferred_element_type=jnp.float32) + bc
    h1 = jnp.maximum(jnp.dot(o1, W1, preferred_element_type=jnp.float32)
                     + b1, 0.0)
    o2 = jnp.dot(h1, W2, preferred_element_type=jnp.float32) + b2
    return o1 + o2


def _model_kernel(allc_ref, sidx_ref,
                  eW_ref, eb_ref, eWq_ref, eWk_ref, eWv_ref, eWc_ref,
                  ebc_ref, eW1_ref, eb1_ref, eW2_ref, eb2_ref,
                  dWq_ref, dWk_ref, dWv_ref, dWc_ref, dbc_ref,
                  dW1_ref, db1_ref, dW2_ref, db2_ref,
                  lW_ref, lb_ref, pW_ref, pb_ref, sW_ref, sb_ref,
                  fW_ref, fb_ref, out_ref):
    allc = allc_ref[0]
    x = jnp.dot(allc.T, eW_ref[...],
                preferred_element_type=jnp.float32) + eb_ref[...]
    for i in range(LE):
        x = _layer_tc(x, eWq_ref[i], eWk_ref[i], eWv_ref[i], eWc_ref[i],
                      ebc_ref[i], eW1_ref[i], eb1_ref[i], eW2_ref[i],
                      eb2_ref[i])
    enc_cur = x[0:1]
    enc_uns = x[1:1 + KS]
    enc_p1 = x[1 + KS:1 + KS + KE]
    enc_p2 = x[1 + KS + KE:NTOK]
    emb_last = jnp.dot(enc_cur, lW_ref[...],
                       preferred_element_type=jnp.float32) + lb_ref[...]
    emb_uns = jnp.dot(enc_uns, sW_ref[...],
                      preferred_element_type=jnp.float32) + sb_ref[...]
    left = jnp.dot(jnp.concatenate([enc_p1, enc_p2], axis=1), pW_ref[...],
                   preferred_element_type=jnp.float32) + pb_ref[...]
    out = jnp.concatenate([emb_last, emb_uns, left], axis=0)
    for i in range(LD):
        out = _layer_tc(out, dWq_ref[i], dWk_ref[i], dWv_ref[i], dWc_ref[i],
                        dbc_ref[i], dW1_ref[i], db1_ref[i], dW2_ref[i],
                        db2_ref[i])
    sl = out[1 + KS:NDEC]
    logits = (jnp.dot(sl, fW_ref[...],
                      preferred_element_type=jnp.float32) + fb_ref[...]).T
    m = jnp.max(logits)
    e = jnp.exp(logits - m)
    props = e / jnp.sum(e)

    sidx = sidx_ref[0]
    p_out = out_ref.shape[2]
    oh = (jax.lax.broadcasted_iota(jnp.int32, (KE, p_out), 1)
          == sidx.T).astype(jnp.float32)
    out_ref[0] = jnp.dot(props, oh, preferred_element_type=jnp.float32)


def kernel(data, abs_partial_solu_2, abs_scatter_solu_1_seleted,
           abs_scatter_solu_1_unseleted, enc_embed_W, enc_embed_b,
           enc_Wq, enc_Wk, enc_Wv, enc_Wc, enc_bc, enc_W1, enc_b1,
           enc_W2, enc_b2, dec_Wq, dec_Wk, dec_Wv, dec_Wc, dec_bc,
           dec_W1, dec_b1, dec_W2, dec_b2, last_W, last_b, part_W,
           part_b, scat_W, scat_b, fin_W, fin_b):
    b = data.shape[0]
    p = abs_partial_solu_2.shape[1]
    u = abs_scatter_solu_1_unseleted.shape[1]
    up = u + 1

    dataT = data.transpose(0, 2, 1)

    def gatherT(idx):
        return jnp.take_along_axis(
            dataT, jnp.broadcast_to(idx[:, None, :], (b, 2, idx.shape[1])),
            axis=2)

    parT = gatherT(abs_partial_solu_2)
    curT = gatherT(abs_scatter_solu_1_seleted)
    unsT = gatherT(abs_scatter_solu_1_unseleted)
    unsT = jnp.pad(unsT, ((0, 0), (0, 0), (0, 1)), constant_values=1e9)

    allc, sidx = pl.pallas_call(
        _prep_kernel,
        grid=(b,),
        in_specs=[
            pl.BlockSpec((1, 2, p), lambda i: (i, 0, 0)),
            pl.BlockSpec((1, 2, up), lambda i: (i, 0, 0)),
            pl.BlockSpec((1, 2, 1), lambda i: (i, 0, 0)),
        ],
        out_specs=[
            pl.BlockSpec((1, 2, NTOK), lambda i: (i, 0, 0)),
            pl.BlockSpec((1, 1, KE), lambda i: (i, 0, 0)),
        ],
        out_shape=[
            jax.ShapeDtypeStruct((b, 2, NTOK), jnp.float32),
            jax.ShapeDtypeStruct((b, 1, KE), jnp.int32),
        ],
    )(parT, unsT, curT)

    w_list = [
        enc_embed_W, enc_embed_b.reshape(1, D),
        enc_Wq, enc_Wk, enc_Wv, enc_Wc, enc_bc.reshape(LE, 1, D),
        enc_W1, enc_b1.reshape(LE, 1, FF), enc_W2, enc_b2.reshape(LE, 1, D),
        dec_Wq, dec_Wk, dec_Wv, dec_Wc, dec_bc.reshape(LD, 1, D),
        dec_W1, dec_b1.reshape(LD, 1, FF), dec_W2, dec_b2.reshape(LD, 1, D),
        last_W, last_b.reshape(1, D), part_W, part_b.reshape(1, D),
        scat_W, scat_b.reshape(1, D), fin_W, fin_b.reshape(1, 1),
    ]

    def const_spec(w):
        nd = w.ndim
        return pl.BlockSpec(w.shape, lambda i, _nd=nd: (0,) * _nd)

    out = pl.pallas_call(
        _model_kernel,
        grid=(b,),
        in_specs=[
            pl.BlockSpec((1, 2, NTOK), lambda i: (i, 0, 0)),
            pl.BlockSpec((1, 1, KE), lambda i: (i, 0, 0)),
        ] + [const_spec(w) for w in w_list],
        out_specs=pl.BlockSpec((1, 1, p), lambda i: (i, 0, 0)),
        out_shape=jax.ShapeDtypeStruct((b, 1, p), jnp.float32),
    )(allc, sidx, *w_list)

    return out.reshape(b, p)
